# baseline (speedup 1.0000x reference)
import functools

import jax
import jax.numpy as jnp
from jax import lax
from jax.experimental import pallas as pl
from jax.experimental.pallas import tpu as pltpu
from jax.experimental.pallas import tpu_sc as plsc

NENTITY = 1000000
NRELATION = 1000
D = 64
B = 16384
L = 16
NC, NS = 2, 16
NW = NC * NS
BPW = B // NW
C = 16
NCHUNK = BPW // C
NBUF = 2
WPC = C * D
NROW = WPC // 128


def _sc_score(heads, rels, tails, ent_flat, rel_flat, gamma_arr):
    mesh = plsc.VectorSubcoreMesh(
        core_axis_name="c", subcore_axis_name="s", num_cores=NC, num_subcores=NS
    )

    @functools.partial(
        pl.kernel,
        out_type=jax.ShapeDtypeStruct((B,), jnp.float32),
        mesh=mesh,
        compiler_params=pltpu.CompilerParams(
            needs_layout_passes=False, use_tc_tiling_on_sc=False
        ),
        scratch_types=dict(
            r_ids=pltpu.VMEM((BPW,), jnp.int32),
            h_idv=pltpu.VMEM((BPW,), jnp.int32),
            t_idv=pltpu.VMEM((BPW,), jnp.int32),
            h_wix=pltpu.VMEM((NBUF, NROW, 128), jnp.int32),
            t_wix=pltpu.VMEM((NBUF, NROW, 128), jnp.int32),
            r_wix=pltpu.VMEM((NBUF, NROW, 128), jnp.int32),
            h_cols=pltpu.VMEM((NBUF, NROW, 128), jnp.float32),
            t_cols=pltpu.VMEM((NBUF, NROW, 128), jnp.float32),
            r_cols=pltpu.VMEM((NBUF, NROW, 128), jnp.float32),
            out_v=pltpu.VMEM((BPW,), jnp.float32),
            gamma_v=pltpu.VMEM((L,), jnp.float32),
            sem0=pltpu.SemaphoreType.DMA,
            sem1=pltpu.SemaphoreType.DMA,
        ),
    )
    def body(heads_hbm, rels_hbm, tails_hbm, ent_hbm, rel_hbm, gamma_hbm,
             out_hbm, r_ids, h_idv, t_idv, h_wix, t_wix, r_wix, h_cols,
             t_cols, r_cols, out_v, gamma_v, sem0, sem1):
        wid = lax.axis_index("s") * NC + lax.axis_index("c")
        base = wid * BPW
        sems = (sem0, sem1)

        pltpu.sync_copy(heads_hbm.at[pl.ds(base, BPW)], h_idv)
        pltpu.sync_copy(tails_hbm.at[pl.ds(base, BPW)], t_idv)
        pltpu.sync_copy(rels_hbm.at[pl.ds(base, BPW)], r_ids)
        pltpu.sync_copy(gamma_hbm, gamma_v)

        def issue(c, buf):
            sem = sems[buf]
            dsl = pl.ds(c * C, C)
            h16 = h_idv[dsl]
            t16 = t_idv[dsl]
            r16 = r_ids[dsl]
            hw = h_wix.at[buf]
            tw = t_wix.at[buf]
            rw = r_wix.at[buf]
            for d in range(D):
                esl = pl.ds((d % 8) * L, L)
                hw[d // 8, esl] = h16 + d * NENTITY
                tw[d // 8, esl] = t16 + d * NENTITY
                rw[d // 8, esl] = r16 * (2 * D) + d
            for r in range(NROW):
                pltpu.async_copy(ent_hbm.at[hw.at[r]], h_cols.at[buf].at[r], sem)
                pltpu.async_copy(ent_hbm.at[tw.at[r]], t_cols.at[buf].at[r], sem)
                pltpu.async_copy(rel_hbm.at[rw.at[r]], r_cols.at[buf].at[r], sem)

        def drain(c, buf):
            sem = sems[buf]
            for r in range(NROW):
                pltpu.make_async_copy(ent_hbm.at[h_wix.at[buf].at[r]],
                                      h_cols.at[buf].at[r], sem).wait()
                pltpu.make_async_copy(ent_hbm.at[t_wix.at[buf].at[r]],
                                      t_cols.at[buf].at[r], sem).wait()
                pltpu.make_async_copy(rel_hbm.at[r_wix.at[buf].at[r]],
                                      r_cols.at[buf].at[r], sem).wait()

        issue(0, 0)
        issue(1, 1)

        gvec = gamma_v[...]

        def chunk_body(half, carry):
            for b in range(NBUF):
                c = half * NBUF + b
                drain(c, b)
                hbuf = h_cols.at[b]
                tbuf = t_cols.at[b]
                rbuf = r_cols.at[b]
                acc = None
                for d in range(D):
                    esl = pl.ds((d % 8) * L, L)
                    term = jnp.abs(hbuf[d // 8, esl] + rbuf[d // 8, esl]
                                   - tbuf[d // 8, esl])
                    acc = term if acc is None else acc + term
                out_v[pl.ds(c * C, C)] = gvec - acc

                @pl.when(c + NBUF < NCHUNK)
                def _():
                    issue(c + NBUF, b)

            return carry

        lax.fori_loop(0, NCHUNK // NBUF, chunk_body, 0)

        pltpu.sync_copy(out_v, out_hbm.at[pl.ds(base, BPW)])

    return body(heads, rels, tails, ent_flat, rel_flat, gamma_arr)


def kernel(sample, entity_embedding, relation_embedding, gamma):
    heads = sample[:, 0]
    rels = sample[:, 1]
    tails = sample[:, 2]
    ent_flat = entity_embedding.T.reshape(-1)
    rel_flat = jnp.pad(relation_embedding, ((0, 0), (0, D))).reshape(-1)
    gamma_arr = jnp.full((L,), gamma, dtype=jnp.float32)
    score = _sc_score(heads, rels, tails, ent_flat, rel_flat, gamma_arr)
    return score.reshape(B, 1)
